# speedup vs baseline: 1.9008x; 1.9008x over previous
import functools

import jax
import jax.numpy as jnp
from jax import lax
from jax.experimental import pallas as pl
from jax.experimental.pallas import tpu as pltpu
from jax.experimental.pallas import tpu_sc as plsc

B = 4096
SEQ = 200
D = 128
CPAD = 16

CH = 104
NCHUNK = 2
LP = CH * NCHUNK

NC, NS = 2, 16
NW = NC * NS
BPW = B // NW

_mesh = plsc.VectorSubcoreMesh(core_axis_name="c", subcore_axis_name="s")


@functools.partial(
    pl.kernel,
    out_type=jax.ShapeDtypeStruct((B, D), jnp.float32),
    mesh=_mesh,
    scratch_types=[
        pltpu.VMEM((BPW, NCHUNK, CH), jnp.int32),
        pltpu.VMEM((LP, D), jnp.float32),
        pltpu.VMEM((LP, D), jnp.float32),
        pltpu.VMEM((BPW, D), jnp.float32),
        pltpu.SemaphoreType.DMA,
        pltpu.SemaphoreType.DMA,
    ],
)
def _pool_sc(x_hbm, table_hbm, out_hbm, idx_v, buf0, buf1, out_v, sem0, sem1):
    wid = lax.axis_index("s") * NC + lax.axis_index("c")
    base = wid * BPW
    pltpu.sync_copy(x_hbm.at[pl.ds(base, BPW)], idx_v)

    bufs = (buf0, buf1)
    sems = (sem0, sem1)

    def fire(b, slot):
        buf, sem = bufs[slot], sems[slot]
        pltpu.async_copy(table_hbm.at[idx_v.at[b, 0]], buf.at[pl.ds(0, CH)], sem)
        pltpu.async_copy(table_hbm.at[idx_v.at[b, 1]], buf.at[pl.ds(CH, CH)], sem)

    def drain(slot):
        buf, sem = bufs[slot], sems[slot]
        pltpu.make_async_copy(table_hbm.at[pl.ds(0, LP)], buf, sem).wait()

    def accumulate(b, slot):
        buf = bufs[slot]
        acc = tuple(jnp.zeros((16,), jnp.float32) for _ in range(8))

        def body(i, acc):
            return tuple(acc[k] + buf[i, pl.ds(k * 16, 16)] for k in range(8))

        acc = lax.fori_loop(0, SEQ, body, acc)
        scale = jnp.float32(1.0 / SEQ)
        for k in range(8):
            out_v[b, pl.ds(k * 16, 16)] = acc[k] * scale

    fire(0, 0)
    fire(1, 1)

    @pl.loop(0, BPW // 2 - 1)
    def _(g):
        b = g * 2
        for slot in range(2):
            drain(slot)
            fire(b + 2 + slot, slot)
            accumulate(b + slot, slot)

    for slot in range(2):
        drain(slot)
        accumulate(BPW - 2 + slot, slot)

    pltpu.sync_copy(out_v, out_hbm.at[pl.ds(base, BPW)])


def _head_body(p_ref, w_ref, b_ref, o_ref):
    o_ref[...] = (
        jnp.dot(p_ref[...], w_ref[...], preferred_element_type=jnp.float32)
        + b_ref[...]
    )


_head = pl.pallas_call(
    _head_body,
    out_shape=jax.ShapeDtypeStruct((B, CPAD), jnp.float32),
)


def kernel(x, table, W, b):
    xp = jnp.pad(x.astype(jnp.int32), ((0, 0), (0, LP - SEQ)))
    xr = xp.reshape(B, NCHUNK, CH)
    pooled = _pool_sc(xr, table)
    wt = jnp.pad(W.T, ((0, 0), (0, CPAD - W.shape[0])))
    bp = jnp.pad(b, (0, CPAD - b.shape[0])).reshape(1, CPAD)
    out = _head(pooled, wt, bp)
    return out[:, : W.shape[0]]


# speedup vs baseline: 1.9055x; 1.0024x over previous
import functools

import jax
import jax.numpy as jnp
from jax import lax
from jax.experimental import pallas as pl
from jax.experimental.pallas import tpu as pltpu
from jax.experimental.pallas import tpu_sc as plsc

B = 4096
SEQ = 200
D = 128
CPAD = 16

CH = 104
NCHUNK = 2
LP = CH * NCHUNK

NC, NS = 2, 16
NW = NC * NS
BPW = B // NW

_mesh = plsc.VectorSubcoreMesh(core_axis_name="c", subcore_axis_name="s")


@functools.partial(
    pl.kernel,
    out_type=jax.ShapeDtypeStruct((B, D), jnp.float32),
    mesh=_mesh,
    scratch_types=[
        pltpu.VMEM((BPW, NCHUNK, CH), jnp.int32),
        pltpu.VMEM((LP, D), jnp.float32),
        pltpu.VMEM((LP, D), jnp.float32),
        pltpu.VMEM((LP, D), jnp.float32),
        pltpu.VMEM((BPW, D), jnp.float32),
        pltpu.SemaphoreType.DMA,
        pltpu.SemaphoreType.DMA,
        pltpu.SemaphoreType.DMA,
    ],
)
def _pool_sc(x_hbm, table_hbm, out_hbm, idx_v, buf0, buf1, buf2, out_v,
             sem0, sem1, sem2):
    wid = lax.axis_index("s") * NC + lax.axis_index("c")
    base = wid * BPW
    pltpu.sync_copy(x_hbm.at[pl.ds(base, BPW)], idx_v)

    bufs = (buf0, buf1, buf2)
    sems = (sem0, sem1, sem2)
    nbuf = len(bufs)

    def fire(b, slot):
        buf, sem = bufs[slot], sems[slot]
        pltpu.async_copy(table_hbm.at[idx_v.at[b, 0]], buf.at[pl.ds(0, CH)], sem)
        pltpu.async_copy(table_hbm.at[idx_v.at[b, 1]], buf.at[pl.ds(CH, CH)], sem)

    def drain(slot):
        buf, sem = bufs[slot], sems[slot]
        pltpu.make_async_copy(table_hbm.at[pl.ds(0, LP)], buf, sem).wait()

    def accumulate(b, slot):
        buf = bufs[slot]
        acc = tuple(jnp.zeros((16,), jnp.float32) for _ in range(8))

        def body(i, acc):
            r = i * 2
            return tuple(
                acc[k] + (buf[r, pl.ds(k * 16, 16)] + buf[r + 1, pl.ds(k * 16, 16)])
                for k in range(8)
            )

        acc = lax.fori_loop(0, SEQ // 2, body, acc)
        scale = jnp.float32(1.0 / SEQ)
        for k in range(8):
            out_v[b, pl.ds(k * 16, 16)] = acc[k] * scale

    for s in range(nbuf):
        fire(s, s)

    main_iters = BPW // nbuf - 1

    @pl.loop(0, main_iters)
    def _(g):
        b0 = g * nbuf
        for s in range(nbuf):
            drain(s)
            accumulate(b0 + s, s)
            fire(b0 + s + nbuf, s)

    for b in range(main_iters * nbuf, BPW):
        s = b % nbuf
        drain(s)
        accumulate(b, s)
        if b + nbuf < BPW:
            fire(b + nbuf, s)

    pltpu.sync_copy(out_v, out_hbm.at[pl.ds(base, BPW)])


def _head_body(p_ref, w_ref, b_ref, o_ref):
    o_ref[...] = (
        jnp.dot(p_ref[...], w_ref[...], preferred_element_type=jnp.float32)
        + b_ref[...]
    )


_head = pl.pallas_call(
    _head_body,
    out_shape=jax.ShapeDtypeStruct((B, CPAD), jnp.float32),
)


def kernel(x, table, W, b):
    xp = jnp.pad(x.astype(jnp.int32), ((0, 0), (0, LP - SEQ)))
    xr = xp.reshape(B, NCHUNK, CH)
    pooled = _pool_sc(xr, table)
    wt = jnp.pad(W.T, ((0, 0), (0, CPAD - W.shape[0])))
    bp = jnp.pad(b, (0, CPAD - b.shape[0])).reshape(1, CPAD)
    out = _head(pooled, wt, bp)
    return out[:, : W.shape[0]]


# speedup vs baseline: 3.1701x; 1.6637x over previous
import functools

import jax
import jax.numpy as jnp
from jax import lax
from jax.experimental import pallas as pl
from jax.experimental.pallas import tpu as pltpu
from jax.experimental.pallas import tpu_sc as plsc

B = 4096
SEQ = 200
D = 128
CPAD = 16

CH = 104
NCHUNK = 2
LP = CH * NCHUNK

NC, NS = 2, 16
NW = NC * NS
BPW = B // NW

_mesh = plsc.VectorSubcoreMesh(core_axis_name="c", subcore_axis_name="s")


@functools.partial(
    pl.kernel,
    out_type=jax.ShapeDtypeStruct((B, D), jnp.bfloat16),
    mesh=_mesh,
    scratch_types=[
        pltpu.VMEM((BPW, NCHUNK, CH), jnp.int32),
        pltpu.VMEM((LP, D), jnp.bfloat16),
        pltpu.VMEM((LP, D), jnp.bfloat16),
        pltpu.VMEM((LP, D), jnp.bfloat16),
        pltpu.VMEM((BPW, D), jnp.bfloat16),
        pltpu.SemaphoreType.DMA,
        pltpu.SemaphoreType.DMA,
        pltpu.SemaphoreType.DMA,
    ],
    compiler_params=pltpu.CompilerParams(
        use_tc_tiling_on_sc=False, needs_layout_passes=False
    ),
)
def _pool_sc(x_hbm, table_hbm, out_hbm, idx_v, buf0, buf1, buf2, out_v,
             sem0, sem1, sem2):
    wid = lax.axis_index("s") * NC + lax.axis_index("c")
    base = wid * BPW
    pltpu.sync_copy(x_hbm.at[pl.ds(base, BPW)], idx_v)

    bufs = (buf0, buf1, buf2)
    sems = (sem0, sem1, sem2)
    nbuf = len(bufs)

    def fire(b, slot):
        buf, sem = bufs[slot], sems[slot]
        pltpu.async_copy(table_hbm.at[idx_v.at[b, 0]], buf.at[pl.ds(0, CH)], sem)
        pltpu.async_copy(table_hbm.at[idx_v.at[b, 1]], buf.at[pl.ds(CH, CH)], sem)

    def drain(slot):
        buf, sem = bufs[slot], sems[slot]
        pltpu.make_async_copy(table_hbm.at[pl.ds(0, LP)], buf, sem).wait()

    def accumulate(b, slot):
        buf = bufs[slot]
        acc = tuple(jnp.zeros((16,), jnp.float32) for _ in range(8))

        def body(i, acc):
            new = []
            for k in range(4):
                lo, hi = plsc.unpack(
                    buf[i, pl.ds(k * 32, 32)], format=plsc.PackFormat.INTERLEAVED
                )
                new.append(acc[2 * k] + lo)
                new.append(acc[2 * k + 1] + hi)
            return tuple(new)

        acc = lax.fori_loop(0, SEQ, body, acc)
        scale = jnp.float32(1.0 / SEQ)
        for k in range(4):
            out_v[b, pl.ds(k * 32, 32)] = plsc.pack(
                acc[2 * k] * scale,
                acc[2 * k + 1] * scale,
                format=plsc.PackFormat.INTERLEAVED,
            )

    for s in range(nbuf):
        fire(s, s)

    main_iters = BPW // nbuf - 1

    @pl.loop(0, main_iters)
    def _(g):
        b0 = g * nbuf
        for s in range(nbuf):
            drain(s)
            accumulate(b0 + s, s)
            fire(b0 + s + nbuf, s)

    for b in range(main_iters * nbuf, BPW):
        s = b % nbuf
        drain(s)
        accumulate(b, s)
        if b + nbuf < BPW:
            fire(b + nbuf, s)

    pltpu.sync_copy(out_v, out_hbm.at[pl.ds(base, BPW)])


def _head_body(p_ref, w_ref, b_ref, o_ref):
    p = p_ref[...].astype(jnp.float32)
    o_ref[...] = (
        jnp.dot(p, w_ref[...], preferred_element_type=jnp.float32) + b_ref[...]
    )


_head = pl.pallas_call(
    _head_body,
    out_shape=jax.ShapeDtypeStruct((B, CPAD), jnp.float32),
)


def kernel(x, table, W, b):
    xp = jnp.pad(x.astype(jnp.int32), ((0, 0), (0, LP - SEQ)))
    xr = xp.reshape(B, NCHUNK, CH)
    pooled = _pool_sc(xr, table.astype(jnp.bfloat16))
    wt = jnp.pad(W.T, ((0, 0), (0, CPAD - W.shape[0])))
    bp = jnp.pad(b, (0, CPAD - b.shape[0])).reshape(1, CPAD)
    out = _head(pooled, wt, bp)
    return out[:, : W.shape[0]]


# speedup vs baseline: 11.6873x; 3.6867x over previous
import functools

import jax
import jax.numpy as jnp
from jax import lax
from jax.experimental import pallas as pl
from jax.experimental.pallas import tpu as pltpu
from jax.experimental.pallas import tpu_sc as plsc

B = 4096
SEQ = 200
D = 128
CPAD = 16

CH0 = 104
CH1 = SEQ - CH0

NC, NS = 2, 16
NW = NC * NS
BPW = B // NW

_mesh = plsc.VectorSubcoreMesh(core_axis_name="c", subcore_axis_name="s")


@functools.partial(
    pl.kernel,
    out_type=jax.ShapeDtypeStruct((B, D), jnp.bfloat16),
    mesh=_mesh,
    scratch_types=[
        pltpu.VMEM((BPW, SEQ), jnp.int32),
        pltpu.VMEM((SEQ, D), jnp.bfloat16),
        pltpu.VMEM((SEQ, D), jnp.bfloat16),
        pltpu.VMEM((SEQ, D), jnp.bfloat16),
        pltpu.VMEM((BPW, D), jnp.bfloat16),
        pltpu.SemaphoreType.DMA,
        pltpu.SemaphoreType.DMA,
        pltpu.SemaphoreType.DMA,
    ],
    compiler_params=pltpu.CompilerParams(
        use_tc_tiling_on_sc=False, needs_layout_passes=False
    ),
)
def _pool_sc(x_hbm, table_hbm, out_hbm, idx_v, buf0, buf1, buf2, out_v,
             sem0, sem1, sem2):
    wid = lax.axis_index("s") * NC + lax.axis_index("c")
    base = wid * BPW

    bufs = (buf0, buf1, buf2)
    sems = (sem0, sem1, sem2)
    nbuf = len(bufs)

    def fire(b, slot):
        buf, sem = bufs[slot], sems[slot]
        pltpu.async_copy(
            table_hbm.at[idx_v.at[b, pl.ds(0, CH0)]], buf.at[pl.ds(0, CH0)], sem
        )
        pltpu.async_copy(
            table_hbm.at[idx_v.at[b, pl.ds(CH0, CH1)]], buf.at[pl.ds(CH0, CH1)], sem
        )

    def drain(slot):
        buf, sem = bufs[slot], sems[slot]
        pltpu.make_async_copy(table_hbm.at[pl.ds(0, SEQ)], buf, sem).wait()

    def accumulate(b, slot):
        buf = bufs[slot]
        acc = tuple(jnp.zeros((16,), jnp.float32) for _ in range(8))

        def body(i, acc):
            new = []
            for k in range(4):
                lo, hi = plsc.unpack(
                    buf[i, pl.ds(k * 32, 32)], format=plsc.PackFormat.INTERLEAVED
                )
                new.append(acc[2 * k] + lo)
                new.append(acc[2 * k + 1] + hi)
            return tuple(new)

        acc = lax.fori_loop(0, SEQ, body, acc)
        scale = jnp.float32(1.0 / SEQ)
        for k in range(4):
            out_v[b, pl.ds(k * 32, 32)] = plsc.pack(
                acc[2 * k] * scale,
                acc[2 * k + 1] * scale,
                format=plsc.PackFormat.INTERLEAVED,
            )

    half = BPW // 2
    pltpu.sync_copy(x_hbm.at[pl.ds(base, half)], idx_v.at[pl.ds(0, half)])
    for s in range(nbuf):
        fire(s, s)
    pltpu.sync_copy(
        x_hbm.at[pl.ds(base + half, half)], idx_v.at[pl.ds(half, half)]
    )

    main_iters = BPW // nbuf - 1

    @pl.loop(0, main_iters)
    def _(g):
        b0 = g * nbuf
        for s in range(nbuf):
            drain(s)
            accumulate(b0 + s, s)
            fire(b0 + s + nbuf, s)

    for b in range(main_iters * nbuf, BPW):
        s = b % nbuf
        drain(s)
        accumulate(b, s)
        if b + nbuf < BPW:
            fire(b + nbuf, s)

    pltpu.sync_copy(out_v, out_hbm.at[pl.ds(base, BPW)])


def _head_body(p_ref, w_ref, b_ref, o_ref):
    p = p_ref[...].astype(jnp.float32)
    o_ref[...] = (
        jnp.dot(p, w_ref[...], preferred_element_type=jnp.float32) + b_ref[...]
    )


_head = pl.pallas_call(
    _head_body,
    out_shape=jax.ShapeDtypeStruct((B, CPAD), jnp.float32),
)


def kernel(x, table, W, b):
    pooled = _pool_sc(x.astype(jnp.int32), table.astype(jnp.bfloat16))
    wt = jnp.pad(W.T, ((0, 0), (0, CPAD - W.shape[0])))
    bp = jnp.pad(b, (0, CPAD - b.shape[0])).reshape(1, CPAD)
    out = _head(pooled, wt, bp)
    return out[:, : W.shape[0]]


# speedup vs baseline: 14.3750x; 1.2300x over previous
import functools

import jax
import jax.numpy as jnp
from jax import lax
from jax.experimental import pallas as pl
from jax.experimental.pallas import tpu as pltpu
from jax.experimental.pallas import tpu_sc as plsc

B = 4096
SEQ = 200
D = 128
VOCAB = 100000
CPAD = 16

CH0 = 104
CH1 = SEQ - CH0

NC, NS = 2, 16
NW = NC * NS
BPW = B // NW

_mesh = plsc.VectorSubcoreMesh(core_axis_name="c", subcore_axis_name="s")


@functools.partial(
    pl.kernel,
    out_type=jax.ShapeDtypeStruct((B, CPAD), jnp.float32),
    mesh=_mesh,
    scratch_types=[
        pltpu.VMEM((BPW, SEQ), jnp.int32),
        pltpu.VMEM((SEQ, CPAD), jnp.float32),
        pltpu.VMEM((SEQ, CPAD), jnp.float32),
        pltpu.VMEM((SEQ, CPAD), jnp.float32),
        pltpu.VMEM((BPW, CPAD), jnp.float32),
        pltpu.VMEM((CPAD,), jnp.float32),
        pltpu.SemaphoreType.DMA,
        pltpu.SemaphoreType.DMA,
        pltpu.SemaphoreType.DMA,
    ],
    compiler_params=pltpu.CompilerParams(
        use_tc_tiling_on_sc=False, needs_layout_passes=False
    ),
)
def _pool_sc(x_hbm, tw_hbm, bias_hbm, out_hbm, idx_v, buf0, buf1, buf2,
             out_v, bias_v, sem0, sem1, sem2):
    wid = lax.axis_index("s") * NC + lax.axis_index("c")
    base = wid * BPW

    bufs = (buf0, buf1, buf2)
    sems = (sem0, sem1, sem2)
    nbuf = len(bufs)

    def fire(b, slot):
        buf, sem = bufs[slot], sems[slot]
        pltpu.async_copy(
            tw_hbm.at[idx_v.at[b, pl.ds(0, CH0)]], buf.at[pl.ds(0, CH0)], sem
        )
        pltpu.async_copy(
            tw_hbm.at[idx_v.at[b, pl.ds(CH0, CH1)]], buf.at[pl.ds(CH0, CH1)], sem
        )

    def drain(slot):
        buf, sem = bufs[slot], sems[slot]
        pltpu.make_async_copy(tw_hbm.at[pl.ds(0, SEQ)], buf, sem).wait()

    def accumulate(b, slot, bias):
        buf = bufs[slot]
        zero = jnp.zeros((16,), jnp.float32)

        def body(i, acc):
            r = i * 4
            return tuple(
                acc[j] + buf[r + j, pl.ds(0, 16)] for j in range(4)
            )

        a0, a1, a2, a3 = lax.fori_loop(0, SEQ // 4, body, (zero,) * 4)
        total = (a0 + a1) + (a2 + a3)
        out_v[b, pl.ds(0, 16)] = total * jnp.float32(1.0 / SEQ) + bias

    pltpu.sync_copy(bias_hbm, bias_v)
    bias = bias_v[pl.ds(0, 16)]

    half = BPW // 2
    pltpu.sync_copy(x_hbm.at[pl.ds(base, half)], idx_v.at[pl.ds(0, half)])
    for s in range(nbuf):
        fire(s, s)
    pltpu.sync_copy(
        x_hbm.at[pl.ds(base + half, half)], idx_v.at[pl.ds(half, half)]
    )

    main_iters = BPW // nbuf - 1

    @pl.loop(0, main_iters)
    def _(g):
        b0 = g * nbuf
        for s in range(nbuf):
            drain(s)
            accumulate(b0 + s, s, bias)
            fire(b0 + s + nbuf, s)

    for b in range(main_iters * nbuf, BPW):
        s = b % nbuf
        drain(s)
        accumulate(b, s, bias)
        if b + nbuf < BPW:
            fire(b + nbuf, s)

    pltpu.sync_copy(out_v, out_hbm.at[pl.ds(base, BPW)])


VB = 1000


def _tw_body(t_ref, w_ref, o_ref):
    o_ref[...] = jnp.dot(
        t_ref[...], w_ref[...], preferred_element_type=jnp.float32
    )


_tw = pl.pallas_call(
    _tw_body,
    grid=(VOCAB // VB,),
    in_specs=[
        pl.BlockSpec((VB, D), lambda i: (i, 0)),
        pl.BlockSpec((D, CPAD), lambda i: (0, 0)),
    ],
    out_specs=pl.BlockSpec((VB, CPAD), lambda i: (i, 0)),
    out_shape=jax.ShapeDtypeStruct((VOCAB, CPAD), jnp.float32),
)


def kernel(x, table, W, b):
    wt = jnp.pad(W.T, ((0, 0), (0, CPAD - W.shape[0])))
    tw = _tw(table, wt)
    bp = jnp.pad(b, (0, CPAD - b.shape[0]))
    out = _pool_sc(x.astype(jnp.int32), tw, bp)
    return out[:, : W.shape[0]]


# speedup vs baseline: 18.4297x; 1.2821x over previous
import functools

import jax
import jax.numpy as jnp
from jax import lax
from jax.experimental import pallas as pl
from jax.experimental.pallas import tpu as pltpu
from jax.experimental.pallas import tpu_sc as plsc

B = 4096
SEQ = 200
D = 128
VOCAB = 100000
CPAD = 16

CH0 = 104
CH1 = SEQ - CH0

NC, NS = 2, 16
NW = NC * NS
BPW = B // NW

_mesh = plsc.VectorSubcoreMesh(core_axis_name="c", subcore_axis_name="s")


@functools.partial(
    pl.kernel,
    out_type=jax.ShapeDtypeStruct((B, CPAD), jnp.float32),
    mesh=_mesh,
    scratch_types=[
        pltpu.VMEM((BPW, SEQ), jnp.int32),
        pltpu.VMEM((SEQ, CPAD), jnp.float32),
        pltpu.VMEM((SEQ, CPAD), jnp.float32),
        pltpu.VMEM((SEQ, CPAD), jnp.float32),
        pltpu.VMEM((BPW, CPAD), jnp.float32),
        pltpu.VMEM((CPAD,), jnp.float32),
        pltpu.SemaphoreType.DMA,
        pltpu.SemaphoreType.DMA,
        pltpu.SemaphoreType.DMA,
    ],
    compiler_params=pltpu.CompilerParams(
        use_tc_tiling_on_sc=False, needs_layout_passes=False
    ),
)
def _pool_sc(x_hbm, tw_hbm, bias_hbm, out_hbm, idx_v, buf0, buf1, buf2,
             out_v, bias_v, sem0, sem1, sem2):
    wid = lax.axis_index("s") * NC + lax.axis_index("c")
    base = wid * BPW

    bufs = (buf0, buf1, buf2)
    sems = (sem0, sem1, sem2)
    nbuf = len(bufs)

    def fire(b, slot):
        buf, sem = bufs[slot], sems[slot]
        pltpu.async_copy(
            tw_hbm.at[idx_v.at[b, pl.ds(0, CH0)]], buf.at[pl.ds(0, CH0)], sem
        )
        pltpu.async_copy(
            tw_hbm.at[idx_v.at[b, pl.ds(CH0, CH1)]], buf.at[pl.ds(CH0, CH1)], sem
        )

    def drain(slot):
        buf, sem = bufs[slot], sems[slot]
        pltpu.make_async_copy(tw_hbm.at[pl.ds(0, SEQ)], buf, sem).wait()

    def accumulate(b, slot, bias):
        buf = bufs[slot]
        zero = jnp.zeros((16,), jnp.float32)

        def body(i, acc):
            r = i * 4
            return tuple(
                acc[j] + buf[r + j, pl.ds(0, 16)] for j in range(4)
            )

        a0, a1, a2, a3 = lax.fori_loop(0, SEQ // 4, body, (zero,) * 4)
        total = (a0 + a1) + (a2 + a3)
        out_v[b, pl.ds(0, 16)] = total * jnp.float32(1.0 / SEQ) + bias

    pltpu.sync_copy(bias_hbm, bias_v)
    bias = bias_v[pl.ds(0, 16)]

    half = BPW // 2
    pltpu.sync_copy(x_hbm.at[pl.ds(base, half)], idx_v.at[pl.ds(0, half)])
    for s in range(nbuf):
        fire(s, s)
    pltpu.sync_copy(
        x_hbm.at[pl.ds(base + half, half)], idx_v.at[pl.ds(half, half)]
    )

    main_iters = BPW // nbuf - 1

    @pl.loop(0, main_iters)
    def _(g):
        b0 = g * nbuf
        for s in range(nbuf):
            drain(s)
            accumulate(b0 + s, s, bias)
            fire(b0 + s + nbuf, s)

    for b in range(main_iters * nbuf, BPW):
        s = b % nbuf
        drain(s)
        accumulate(b, s, bias)
        if b + nbuf < BPW:
            fire(b + nbuf, s)

    pltpu.sync_copy(out_v, out_hbm.at[pl.ds(base, BPW)])


VB = 5000


def _tw_body(t_ref, w_ref, o_ref):
    o_ref[...] = jnp.dot(
        t_ref[...], w_ref[...], preferred_element_type=jnp.float32
    )


_tw = pl.pallas_call(
    _tw_body,
    grid=(VOCAB // VB,),
    in_specs=[
        pl.BlockSpec((VB, D), lambda i: (i, 0)),
        pl.BlockSpec((D, CPAD), lambda i: (0, 0)),
    ],
    out_specs=pl.BlockSpec((VB, CPAD), lambda i: (i, 0)),
    out_shape=jax.ShapeDtypeStruct((VOCAB, CPAD), jnp.float32),
)


def kernel(x, table, W, b):
    wt = jnp.pad(W.T, ((0, 0), (0, CPAD - W.shape[0])))
    tw = _tw(table, wt)
    bp = jnp.pad(b, (0, CPAD - b.shape[0]))
    if x.dtype != jnp.int32:
        x = x.astype(jnp.int32)
    out = _pool_sc(x, tw, bp)
    return out[:, : W.shape[0]]


# speedup vs baseline: 19.0664x; 1.0345x over previous
import functools

import jax
import jax.numpy as jnp
from jax import lax
from jax.experimental import pallas as pl
from jax.experimental.pallas import tpu as pltpu
from jax.experimental.pallas import tpu_sc as plsc

B = 4096
SEQ = 200
D = 128
VOCAB = 100000
CPAD = 16

CH0 = 104
CH1 = SEQ - CH0

NC, NS = 2, 16
NW = NC * NS
BPW = B // NW

_mesh = plsc.VectorSubcoreMesh(core_axis_name="c", subcore_axis_name="s")


@functools.partial(
    pl.kernel,
    out_type=jax.ShapeDtypeStruct((B, CPAD), jnp.float32),
    mesh=_mesh,
    scratch_types=[
        pltpu.VMEM((BPW, SEQ), jnp.int32),
        pltpu.VMEM((SEQ, CPAD), jnp.float32),
        pltpu.VMEM((SEQ, CPAD), jnp.float32),
        pltpu.VMEM((SEQ, CPAD), jnp.float32),
        pltpu.VMEM((BPW, CPAD), jnp.float32),
        pltpu.VMEM((CPAD,), jnp.float32),
        pltpu.SemaphoreType.DMA,
        pltpu.SemaphoreType.DMA,
        pltpu.SemaphoreType.DMA,
    ],
    compiler_params=pltpu.CompilerParams(
        use_tc_tiling_on_sc=False, needs_layout_passes=False
    ),
)
def _pool_sc(x_hbm, tw_hbm, bias_hbm, out_hbm, idx_v, buf0, buf1, buf2,
             out_v, bias_v, sem0, sem1, sem2):
    wid = lax.axis_index("s") * NC + lax.axis_index("c")
    base = wid * BPW

    bufs = (buf0, buf1, buf2)
    sems = (sem0, sem1, sem2)
    nbuf = len(bufs)

    def fire(b, slot):
        buf, sem = bufs[slot], sems[slot]
        pltpu.async_copy(
            tw_hbm.at[idx_v.at[b, pl.ds(0, CH0)]], buf.at[pl.ds(0, CH0)], sem
        )
        pltpu.async_copy(
            tw_hbm.at[idx_v.at[b, pl.ds(CH0, CH1)]], buf.at[pl.ds(CH0, CH1)], sem
        )

    def drain(slot):
        buf, sem = bufs[slot], sems[slot]
        pltpu.make_async_copy(tw_hbm.at[pl.ds(0, SEQ)], buf, sem).wait()

    def accumulate(b, slot, bias):
        buf = bufs[slot]
        zero = jnp.zeros((16,), jnp.float32)

        def body(i, acc):
            r = i * 4
            return tuple(
                acc[j] + buf[r + j, pl.ds(0, 16)] for j in range(4)
            )

        a0, a1, a2, a3 = lax.fori_loop(0, SEQ // 4, body, (zero,) * 4)
        total = (a0 + a1) + (a2 + a3)
        out_v[b, pl.ds(0, 16)] = total * jnp.float32(1.0 / SEQ) + bias

    pltpu.sync_copy(bias_hbm, bias_v)
    bias = bias_v[pl.ds(0, 16)]

    half = BPW // 2
    pltpu.sync_copy(x_hbm.at[pl.ds(base, half)], idx_v.at[pl.ds(0, half)])
    for s in range(nbuf):
        fire(s, s)
    pltpu.sync_copy(
        x_hbm.at[pl.ds(base + half, half)], idx_v.at[pl.ds(half, half)]
    )

    main_iters = BPW // nbuf - 1

    @pl.loop(0, main_iters)
    def _(g):
        b0 = g * nbuf
        for s in range(nbuf):
            drain(s)
            accumulate(b0 + s, s, bias)
            fire(b0 + s + nbuf, s)

    for b in range(main_iters * nbuf, BPW):
        s = b % nbuf
        drain(s)
        accumulate(b, s, bias)
        if b + nbuf < BPW:
            fire(b + nbuf, s)

    pltpu.sync_copy(out_v, out_hbm.at[pl.ds(base, BPW)])


VB = 10000


def _tw_body(t_ref, w_ref, o_ref):
    o_ref[...] = jnp.dot(
        t_ref[...], w_ref[...], preferred_element_type=jnp.float32
    )


_tw = pl.pallas_call(
    _tw_body,
    grid=(VOCAB // VB,),
    in_specs=[
        pl.BlockSpec((VB, D), lambda i: (i, 0)),
        pl.BlockSpec((D, CPAD), lambda i: (0, 0)),
    ],
    out_specs=pl.BlockSpec((VB, CPAD), lambda i: (i, 0)),
    out_shape=jax.ShapeDtypeStruct((VOCAB, CPAD), jnp.float32),
)


def kernel(x, table, W, b):
    wt = jnp.pad(W.T, ((0, 0), (0, CPAD - W.shape[0])))
    tw = _tw(table, wt)
    bp = jnp.pad(b, (0, CPAD - b.shape[0]))
    if x.dtype != jnp.int32:
        x = x.astype(jnp.int32)
    out = _pool_sc(x, tw, bp)
    return out[:, : W.shape[0]]


# speedup vs baseline: 19.0959x; 1.0015x over previous
import functools

import jax
import jax.numpy as jnp
from jax import lax
from jax.experimental import pallas as pl
from jax.experimental.pallas import tpu as pltpu
from jax.experimental.pallas import tpu_sc as plsc

B = 4096
SEQ = 200
D = 128
VOCAB = 100000
CPAD = 16

CH0 = 104
CH1 = SEQ - CH0

NC, NS = 2, 16
NW = NC * NS
BPW = B // NW

_mesh = plsc.VectorSubcoreMesh(core_axis_name="c", subcore_axis_name="s")


@functools.partial(
    pl.kernel,
    out_type=jax.ShapeDtypeStruct((B, CPAD), jnp.float32),
    mesh=_mesh,
    scratch_types=[
        pltpu.VMEM((BPW, SEQ), jnp.int32),
        pltpu.VMEM((SEQ, CPAD), jnp.float32),
        pltpu.VMEM((SEQ, CPAD), jnp.float32),
        pltpu.VMEM((SEQ, CPAD), jnp.float32),
        pltpu.VMEM((BPW, CPAD), jnp.float32),
        pltpu.VMEM((CPAD,), jnp.float32),
        pltpu.SemaphoreType.DMA,
        pltpu.SemaphoreType.DMA,
        pltpu.SemaphoreType.DMA,
    ],
    compiler_params=pltpu.CompilerParams(use_tc_tiling_on_sc=False),
)
def _pool_sc(x_hbm, tw_hbm, bias_hbm, out_hbm, idx_v, buf0, buf1, buf2,
             out_v, bias_v, sem0, sem1, sem2):
    wid = lax.axis_index("s") * NC + lax.axis_index("c")
    base = wid * BPW

    bufs = (buf0, buf1, buf2)
    sems = (sem0, sem1, sem2)
    nbuf = len(bufs)

    def fire(b, slot):
        buf, sem = bufs[slot], sems[slot]
        pltpu.async_copy(
            tw_hbm.at[idx_v.at[b, pl.ds(0, CH0)]], buf.at[pl.ds(0, CH0)], sem
        )
        pltpu.async_copy(
            tw_hbm.at[idx_v.at[b, pl.ds(CH0, CH1)]], buf.at[pl.ds(CH0, CH1)], sem
        )

    def drain(slot):
        buf, sem = bufs[slot], sems[slot]
        pltpu.make_async_copy(tw_hbm.at[pl.ds(0, SEQ)], buf, sem).wait()

    def accumulate(b, slot, bias):
        buf = bufs[slot]
        zero = jnp.zeros((16,), jnp.float32)

        def body(i, acc):
            r = i * 4
            return tuple(
                acc[j] + buf[r + j, pl.ds(0, 16)] for j in range(4)
            )

        a0, a1, a2, a3 = lax.fori_loop(0, SEQ // 4, body, (zero,) * 4)
        total = (a0 + a1) + (a2 + a3)
        out_v[b, pl.ds(0, 16)] = total * jnp.float32(1.0 / SEQ) + bias

    pltpu.sync_copy(bias_hbm, bias_v)
    bias = bias_v[pl.ds(0, 16)]

    half = BPW // 2
    pltpu.sync_copy(x_hbm.at[pl.ds(base, half)], idx_v.at[pl.ds(0, half)])
    for s in range(nbuf):
        fire(s, s)
    pltpu.sync_copy(
        x_hbm.at[pl.ds(base + half, half)], idx_v.at[pl.ds(half, half)]
    )

    main_iters = BPW // nbuf - 1

    @pl.loop(0, main_iters)
    def _(g):
        b0 = g * nbuf
        for s in range(nbuf):
            drain(s)
            accumulate(b0 + s, s, bias)
            fire(b0 + s + nbuf, s)

    for b in range(main_iters * nbuf, BPW):
        s = b % nbuf
        drain(s)
        accumulate(b, s, bias)
        if b + nbuf < BPW:
            fire(b + nbuf, s)

    pltpu.sync_copy(out_v, out_hbm.at[pl.ds(base, BPW)])


VB = 10000


def _tw_body(t_ref, w_ref, o_ref):
    o_ref[...] = jnp.dot(
        t_ref[...], w_ref[...], preferred_element_type=jnp.float32
    )


_tw = pl.pallas_call(
    _tw_body,
    grid=(VOCAB // VB,),
    in_specs=[
        pl.BlockSpec((VB, D), lambda i: (i, 0)),
        pl.BlockSpec((D, CPAD), lambda i: (0, 0)),
    ],
    out_specs=pl.BlockSpec((VB, CPAD), lambda i: (i, 0)),
    out_shape=jax.ShapeDtypeStruct((VOCAB, CPAD), jnp.float32),
)


def kernel(x, table, W, b):
    wt = jnp.pad(W.T, ((0, 0), (0, CPAD - W.shape[0])))
    tw = _tw(table, wt)
    bp = jnp.pad(b, (0, CPAD - b.shape[0]))
    if x.dtype != jnp.int32:
        x = x.astype(jnp.int32)
    out = _pool_sc(x, tw, bp)
    return out[:, : W.shape[0]]
